# baseline (speedup 1.0000x reference)
import jax
import jax.numpy as jnp
from jax.experimental import pallas as pl

_DIM = 32
_PACK = 4
_LANES = _PACK * _DIM
_BLK = 1000

_INV_2PI = 0.15915494309189535
_C0 = 0.99999944368
_C1 = -19.739034373
_C2 = 64.93061337
_C3 = -85.295970962
_C4 = 58.912555324
_C5 = -21.283021593


def _cos2pi(t):
    u = t * t
    return _C0 + u * (_C1 + u * (_C2 + u * (_C3 + u * (_C4 + u * _C5))))


def _gru_body(mts_ref, memts_ref, mail_ref, mem_ref, rh_ref,
              bw_ref, wih_ref, whh_ref, bih_ref, bhh_ref, tb_ref,
              out_ref):
    d = _LANES
    dim = _DIM
    p = _PACK
    dt = jnp.concatenate([mts_ref[g] - memts_ref[g] for g in range(p)],
                         axis=1)
    x = jnp.dot(dt, bw_ref[...], precision=jax.lax.Precision.HIGHEST,
                preferred_element_type=jnp.float32) + tb_ref[...]
    t = x * _INV_2PI
    t = t - jnp.round(t)
    tf = _cos2pi(t)
    mail_p = jnp.concatenate([mail_ref[g] for g in range(p)], axis=1)
    mem_p = jnp.concatenate([mem_ref[g] for g in range(p)], axis=1)
    t_in = jnp.concatenate(
        [mail_p.astype(jnp.bfloat16), tf.astype(jnp.bfloat16)], axis=1)
    gx = jnp.dot(t_in, wih_ref[...],
                 preferred_element_type=jnp.float32) + bih_ref[...]
    gh = jnp.dot(mem_p.astype(jnp.bfloat16), whh_ref[...],
                 preferred_element_type=jnp.float32) + bhh_ref[...]
    r = jax.nn.sigmoid(gx[:, 0:d] + gh[:, 0:d])
    z = jax.nn.sigmoid(gx[:, d:2 * d] + gh[:, d:2 * d])
    n = jnp.tanh(gx[:, 2 * d:3 * d] + r * gh[:, 2 * d:3 * d])
    h = (1.0 - z) * n + z * mem_p
    for g in range(p):
        out_ref[g] = h[:, g * dim:(g + 1) * dim] + rh_ref[g]


def kernel(mail, mail_ts, mem_ts, mem, rh, W_ih, W_hh, b_ih, b_hh, time_w, time_b):
    n = mail.shape[0]
    d = _DIM
    p = _PACK
    lanes = _LANES
    np_ = n // p
    grid = (np_ // _BLK,)

    mail3 = mail.reshape(p, np_, d)
    mem3 = mem.reshape(p, np_, d)
    rh3 = rh.reshape(p, np_, d)
    mts3 = mail_ts.reshape(p, np_, 1)
    memts3 = mem_ts.reshape(p, np_, 1)

    eye = jnp.eye(p, dtype=jnp.float32)
    bw = jnp.kron(eye, time_w.reshape(1, d))
    tb4 = jnp.tile(time_b, p).reshape(1, lanes)

    wih_t = W_ih.T.reshape(2, d, 3, d)
    wih4 = jnp.einsum('pitj,gh->pgithj', wih_t, eye)
    wih4 = wih4.reshape(2 * lanes, 3 * lanes).astype(jnp.bfloat16)
    whh_t = W_hh.T.reshape(d, 3, d)
    whh4 = jnp.einsum('itj,gh->githj', whh_t, eye)
    whh4 = whh4.reshape(lanes, 3 * lanes).astype(jnp.bfloat16)
    bih4 = jnp.broadcast_to(b_ih.reshape(3, 1, d), (3, p, d)).reshape(1, 3 * lanes)
    bhh4 = jnp.broadcast_to(b_hh.reshape(3, 1, d), (3, p, d)).reshape(1, 3 * lanes)

    grp_spec = lambda w: pl.BlockSpec((p, _BLK, w), lambda i: (0, i, 0))
    full_spec = lambda a: pl.BlockSpec(a.shape, lambda i: (0, 0))

    out = pl.pallas_call(
        _gru_body,
        grid=grid,
        in_specs=[
            grp_spec(1),
            grp_spec(1),
            grp_spec(d),
            grp_spec(d),
            grp_spec(d),
            full_spec(bw),
            full_spec(wih4),
            full_spec(whh4),
            full_spec(bih4),
            full_spec(bhh4),
            full_spec(tb4),
        ],
        out_specs=grp_spec(d),
        out_shape=jax.ShapeDtypeStruct((p, np_, d), jnp.float32),
    )(mts3, memts3, mail3, mem3, rh3, bw, wih4, whh4, bih4, bhh4, tb4)
    return out.reshape(n, d)


# speedup vs baseline: 1.2798x; 1.2798x over previous
import jax
import jax.numpy as jnp
from jax.experimental import pallas as pl

_DIM = 32
_PACK = 4
_LANES = _PACK * _DIM
_CHUNK = 1000
_BLK = _PACK * _CHUNK

_INV_2PI = 0.15915494309189535
_C0 = 0.99999944368
_C1 = -19.739034373
_C2 = 64.93061337
_C3 = -85.295970962
_C4 = 58.912555324
_C5 = -21.283021593


def _cos2pi(t):
    u = t * t
    return _C0 + u * (_C1 + u * (_C2 + u * (_C3 + u * (_C4 + u * _C5))))


def _pack_lanes(ref, dtype=None):
    parts = []
    for g in range(_PACK):
        x = ref[g * _CHUNK:(g + 1) * _CHUNK, :]
        parts.append(x if dtype is None else x.astype(dtype))
    return jnp.concatenate(parts, axis=1)


def _gru_body(mts_ref, memts_ref, mail_ref, mem_ref, rh_ref,
              bw_ref, wih_ref, whh_ref, bih_ref, bhh_ref, tb_ref,
              out_ref):
    d = _LANES
    dim = _DIM
    c = _CHUNK
    x = jnp.dot(mts_ref[...] - memts_ref[...], bw_ref[...],
                precision=jax.lax.Precision.HIGHEST,
                preferred_element_type=jnp.float32) + tb_ref[...]
    t = x * _INV_2PI
    t = t - jnp.round(t)
    tf = _cos2pi(t)
    mail_p = _pack_lanes(mail_ref, jnp.bfloat16)
    mem_p = _pack_lanes(mem_ref)
    t_in = jnp.concatenate([mail_p, tf.astype(jnp.bfloat16)], axis=1)
    gx = jnp.dot(t_in, wih_ref[...],
                 preferred_element_type=jnp.float32) + bih_ref[...]
    gh = jnp.dot(mem_p.astype(jnp.bfloat16), whh_ref[...],
                 preferred_element_type=jnp.float32) + bhh_ref[...]
    r = jax.nn.sigmoid(gx[:, 0:d] + gh[:, 0:d])
    z = jax.nn.sigmoid(gx[:, d:2 * d] + gh[:, d:2 * d])
    n = jnp.tanh(gx[:, 2 * d:3 * d] + r * gh[:, 2 * d:3 * d])
    h = (1.0 - z) * n + z * mem_p
    for g in range(_PACK):
        out_ref[g * c:(g + 1) * c, :] = (
            h[:, g * dim:(g + 1) * dim] + rh_ref[g * c:(g + 1) * c, :])


def kernel(mail, mail_ts, mem_ts, mem, rh, W_ih, W_hh, b_ih, b_hh, time_w, time_b):
    n = mail.shape[0]
    d = _DIM
    p = _PACK
    c = _CHUNK
    lanes = _LANES
    grid = (n // _BLK,)

    regroup = lambda ts: ts.reshape(n // _BLK, p, c).transpose(0, 2, 1).reshape(n // p, p)
    mtsc = regroup(mail_ts)
    memtsc = regroup(mem_ts)

    eye = jnp.eye(p, dtype=jnp.float32)
    bw = jnp.kron(eye, time_w.reshape(1, d))
    tb4 = jnp.tile(time_b, p).reshape(1, lanes)

    wih_t = W_ih.T.reshape(2, d, 3, d)
    wih4 = jnp.einsum('pitj,gh->pgithj', wih_t, eye)
    wih4 = wih4.reshape(2 * lanes, 3 * lanes).astype(jnp.bfloat16)
    whh_t = W_hh.T.reshape(d, 3, d)
    whh4 = jnp.einsum('itj,gh->githj', whh_t, eye)
    whh4 = whh4.reshape(lanes, 3 * lanes).astype(jnp.bfloat16)
    bih4 = jnp.broadcast_to(b_ih.reshape(3, 1, d), (3, p, d)).reshape(1, 3 * lanes)
    bhh4 = jnp.broadcast_to(b_hh.reshape(3, 1, d), (3, p, d)).reshape(1, 3 * lanes)

    full_spec = lambda a: pl.BlockSpec(a.shape, lambda i: (0, 0))

    return pl.pallas_call(
        _gru_body,
        grid=grid,
        in_specs=[
            pl.BlockSpec((c, p), lambda i: (i, 0)),
            pl.BlockSpec((c, p), lambda i: (i, 0)),
            pl.BlockSpec((_BLK, d), lambda i: (i, 0)),
            pl.BlockSpec((_BLK, d), lambda i: (i, 0)),
            pl.BlockSpec((_BLK, d), lambda i: (i, 0)),
            full_spec(bw),
            full_spec(wih4),
            full_spec(whh4),
            full_spec(bih4),
            full_spec(bhh4),
            full_spec(tb4),
        ],
        out_specs=pl.BlockSpec((_BLK, d), lambda i: (i, 0)),
        out_shape=jax.ShapeDtypeStruct((n, d), jnp.float32),
    )(mtsc, memtsc, mail, mem, rh, bw, wih4, whh4, bih4, bhh4, tb4)


# speedup vs baseline: 1.3666x; 1.0678x over previous
import jax
import jax.numpy as jnp
from jax.experimental import pallas as pl

_DIM = 32
_PACK = 4
_LANES = _PACK * _DIM
_CHUNK = 1000
_BLK = _PACK * _CHUNK

_INV_2PI = 0.15915494309189535
_C0 = 0.99999944368
_C1 = -19.739034373
_C2 = 64.93061337
_C3 = -85.295970962
_C4 = 58.912555324
_C5 = -21.283021593


def _cos2pi(t):
    u = t * t
    return _C0 + u * (_C1 + u * (_C2 + u * (_C3 + u * (_C4 + u * _C5))))


def _pack_lanes(ref, dtype=None):
    parts = []
    for g in range(_PACK):
        x = ref[g * _CHUNK:(g + 1) * _CHUNK, :]
        parts.append(x if dtype is None else x.astype(dtype))
    return jnp.concatenate(parts, axis=1)


def _gru_body(mts_ref, memts_ref, mail_ref, mem_ref, rh_ref,
              bw_ref, wih_ref, whh_ref, bih_ref, bhh_ref, tb_ref,
              out_ref):
    d = _LANES
    dim = _DIM
    c = _CHUNK
    dt4 = mts_ref[0] - memts_ref[0]
    x = jax.lax.dot_general(
        dt4, bw_ref[...], (((0,), (0,)), ((), ())),
        precision=jax.lax.Precision.HIGHEST,
        preferred_element_type=jnp.float32) + tb_ref[...]
    t = x * _INV_2PI
    t = t - jnp.round(t)
    tf = _cos2pi(t)
    mail_p = _pack_lanes(mail_ref, jnp.bfloat16)
    mem_p = _pack_lanes(mem_ref)
    t_in = jnp.concatenate([mail_p, tf.astype(jnp.bfloat16)], axis=1)
    gx = jnp.dot(t_in, wih_ref[...],
                 preferred_element_type=jnp.float32) + bih_ref[...]
    gh = jnp.dot(mem_p.astype(jnp.bfloat16), whh_ref[...],
                 preferred_element_type=jnp.float32) + bhh_ref[...]
    r = jax.nn.sigmoid(gx[:, 0:d] + gh[:, 0:d])
    z = jax.nn.sigmoid(gx[:, d:2 * d] + gh[:, d:2 * d])
    n = jnp.tanh(gx[:, 2 * d:3 * d] + r * gh[:, 2 * d:3 * d])
    h = (1.0 - z) * n + z * mem_p
    for g in range(_PACK):
        out_ref[g * c:(g + 1) * c, :] = (
            h[:, g * dim:(g + 1) * dim] + rh_ref[g * c:(g + 1) * c, :])


def kernel(mail, mail_ts, mem_ts, mem, rh, W_ih, W_hh, b_ih, b_hh, time_w, time_b):
    n = mail.shape[0]
    d = _DIM
    p = _PACK
    c = _CHUNK
    lanes = _LANES
    grid = (n // _BLK,)

    mtsc = mail_ts.reshape(n // _BLK, p, c)
    memtsc = mem_ts.reshape(n // _BLK, p, c)

    eye = jnp.eye(p, dtype=jnp.float32)
    bw = jnp.kron(eye, time_w.reshape(1, d))
    tb4 = jnp.tile(time_b, p).reshape(1, lanes)

    wih_t = W_ih.T.reshape(2, d, 3, d)
    wih4 = jnp.einsum('pitj,gh->pgithj', wih_t, eye)
    wih4 = wih4.reshape(2 * lanes, 3 * lanes).astype(jnp.bfloat16)
    whh_t = W_hh.T.reshape(d, 3, d)
    whh4 = jnp.einsum('itj,gh->githj', whh_t, eye)
    whh4 = whh4.reshape(lanes, 3 * lanes).astype(jnp.bfloat16)
    bih4 = jnp.broadcast_to(b_ih.reshape(3, 1, d), (3, p, d)).reshape(1, 3 * lanes)
    bhh4 = jnp.broadcast_to(b_hh.reshape(3, 1, d), (3, p, d)).reshape(1, 3 * lanes)

    full_spec = lambda a: pl.BlockSpec(a.shape, lambda i: (0, 0))

    return pl.pallas_call(
        _gru_body,
        grid=grid,
        in_specs=[
            pl.BlockSpec((1, p, c), lambda i: (i, 0, 0)),
            pl.BlockSpec((1, p, c), lambda i: (i, 0, 0)),
            pl.BlockSpec((_BLK, d), lambda i: (i, 0)),
            pl.BlockSpec((_BLK, d), lambda i: (i, 0)),
            pl.BlockSpec((_BLK, d), lambda i: (i, 0)),
            full_spec(bw),
            full_spec(wih4),
            full_spec(whh4),
            full_spec(bih4),
            full_spec(bhh4),
            full_spec(tb4),
        ],
        out_specs=pl.BlockSpec((_BLK, d), lambda i: (i, 0)),
        out_shape=jax.ShapeDtypeStruct((n, d), jnp.float32),
    )(mtsc, memtsc, mail, mem, rh, bw, wih4, whh4, bih4, bhh4, tb4)
